# baseline (speedup 1.0000x reference)
import functools

import jax
import jax.numpy as jnp
from jax import lax
from jax.experimental import pallas as pl
from jax.experimental.pallas import tpu as pltpu
from jax.experimental.pallas import tpu_sc as plsc

_NUM_GROUPS = 8
_K = 1024
_EMBED = 512
_GROUP_DIM = _EMBED // _NUM_GROUPS
_BETA = 4.0
_TILE = 512


def _dist_body(x_ref, cb_ref, pw_ref, pb_ref, idx_ref, part_ref):
    x = x_ref[...]
    loss_acc = jnp.float32(0.0)
    for g in range(_NUM_GROUPS):
        xg = x[:, g * _GROUP_DIM:(g + 1) * _GROUP_DIM]
        p = jnp.dot(xg, pw_ref[g], preferred_element_type=jnp.float32)
        p = p + pb_ref[g][None, :]
        c = cb_ref[g]
        cross2 = jax.lax.dot_general(
            p * jnp.float32(-2.0), c, (((1,), (1,)), ((), ())),
            preferred_element_type=jnp.float32)
        p2 = jnp.sum(p * p, axis=-1, keepdims=True)
        c2 = jnp.sum(c * c, axis=-1)
        dist = (p2 + cross2) + c2[None, :]
        minv = jnp.min(dist, axis=-1)
        loss_acc = loss_acc + jnp.sum(minv)
        eq = dist == minv[:, None]
        iota = jax.lax.broadcasted_iota(jnp.int32, dist.shape, 1)
        idx = jnp.min(jnp.where(eq, iota, _K), axis=-1)
        idx_ref[g, :] = idx
    part_ref[0, 0, 0] = loss_acc


@jax.jit
def _dist_call(x, codebooks, proj_w, proj_b):
    n = x.shape[0]
    grid = n // _TILE
    idx_gm, partials = pl.pallas_call(
        _dist_body,
        grid=(grid,),
        in_specs=[
            pl.BlockSpec((_TILE, _EMBED), lambda i: (i, 0)),
            pl.BlockSpec((_NUM_GROUPS, _K, _GROUP_DIM), lambda i: (0, 0, 0)),
            pl.BlockSpec((_NUM_GROUPS, _GROUP_DIM, _GROUP_DIM),
                         lambda i: (0, 0, 0)),
            pl.BlockSpec((_NUM_GROUPS, _GROUP_DIM), lambda i: (0, 0)),
        ],
        out_specs=[
            pl.BlockSpec((_NUM_GROUPS, _TILE), lambda i: (0, i)),
            pl.BlockSpec((1, 1, 1), lambda i: (i, 0, 0),
                         memory_space=pltpu.SMEM),
        ],
        out_shape=[
            jax.ShapeDtypeStruct((_NUM_GROUPS, n), jnp.int32),
            jax.ShapeDtypeStruct((grid, 1, 1), jnp.float32),
        ],
    )(x, codebooks, proj_w, proj_b)
    return idx_gm, partials


def _make_sc_gather(n_rows):
    info = plsc.get_sparse_core_info()
    nw = info.num_cores * info.num_subcores
    b_per_w = n_rows // nw
    n_chunks = 4
    chunk = b_per_w // n_chunks
    mesh = plsc.VectorSubcoreMesh(core_axis_name="c", subcore_axis_name="s")

    @functools.partial(
        pl.kernel, mesh=mesh,
        compiler_params=pltpu.CompilerParams(use_tc_tiling_on_sc=False),
        out_type=jax.ShapeDtypeStruct((n_rows, _GROUP_DIM), jnp.float32),
        scratch_types=[
            pltpu.VMEM((b_per_w,), jnp.int32),
            pltpu.VMEM((chunk, _GROUP_DIM), jnp.float32),
            pltpu.SemaphoreType.DMA,
        ],
    )
    def gather_kernel(table_hbm, ids_hbm, out_hbm, ids_v, rows_v, sem):
        wid = lax.axis_index("s") * info.num_cores + lax.axis_index("c")
        base = wid * b_per_w
        pltpu.sync_copy(ids_hbm.at[pl.ds(base, b_per_w)], ids_v)
        for ci in range(n_chunks):
            pltpu.async_copy(
                table_hbm.at[ids_v.at[pl.ds(ci * chunk, chunk)]],
                rows_v, sem).wait()
            pltpu.sync_copy(
                rows_v, out_hbm.at[pl.ds(base + ci * chunk, chunk)])

    return gather_kernel


def _out_body(q_ref, ow_ref, ob_ref, out_ref):
    out = jnp.dot(q_ref[...], ow_ref[...],
                  preferred_element_type=jnp.float32)
    out_ref[...] = out + ob_ref[...]


_CHUNKS = 2


@jax.jit
def _pipeline(features, codebooks, proj_w, proj_b, out_w, out_b):
    b, s, e = features.shape
    n = b * s
    nc = n // _CHUNKS
    x = features.reshape(n, e)
    table = codebooks.reshape(_NUM_GROUPS * _K, _GROUP_DIM)
    offs = (jnp.arange(_NUM_GROUPS, dtype=jnp.int32) * _K)[:, None]
    ob2d = out_b.reshape(1, e)

    idx_parts, id_parts, part_parts = [], [], []
    for ch in range(_CHUNKS):
        idx_gm, partials = _dist_call(
            x[ch * nc:(ch + 1) * nc], codebooks, proj_w, proj_b)
        idx_parts.append(idx_gm)
        id_parts.append((idx_gm + offs).T.reshape(nc * _NUM_GROUPS))
        part_parts.append(partials)
    outs = []
    for ch in range(_CHUNKS):
        rows = _make_sc_gather(nc * _NUM_GROUPS)(table, id_parts[ch])
        q = rows.reshape(nc, e)
        grid = nc // _TILE
        out = pl.pallas_call(
            _out_body,
            grid=(grid,),
            in_specs=[
                pl.BlockSpec((_TILE, _EMBED), lambda i: (i, 0)),
                pl.BlockSpec((_EMBED, _EMBED), lambda i: (0, 0)),
                pl.BlockSpec((1, _EMBED), lambda i: (0, 0)),
            ],
            out_specs=pl.BlockSpec((_TILE, _EMBED), lambda i: (i, 0)),
            out_shape=jax.ShapeDtypeStruct((nc, _EMBED), jnp.float32),
        )(q, out_w, ob2d)
        outs.append(out)

    quantized_features = jnp.concatenate(outs, axis=0).reshape(b, s, e)
    indices = jnp.concatenate(idx_parts, axis=1).T.reshape(b, s, _NUM_GROUPS)
    scale = _BETA / (_NUM_GROUPS * b * s * _GROUP_DIM)
    total_commitment_loss = (jnp.sum(jnp.stack(part_parts)) * scale)
    return (quantized_features, indices, total_commitment_loss)


def kernel(features, codebooks, proj_w, proj_b, out_w, out_b):
    return _pipeline(features, codebooks, proj_w, proj_b, out_w, out_b)


# speedup vs baseline: 1.5489x; 1.5489x over previous
import jax
import jax.numpy as jnp
from jax.experimental import pallas as pl
from jax.experimental.pallas import tpu as pltpu

_NUM_GROUPS = 8
_K = 1024
_EMBED = 512
_GROUP_DIM = _EMBED // _NUM_GROUPS
_BETA = 4.0
_TILE = 768
_AUG = 128


def _vq_body(x_ref, cb_ref, pw_ref, pb_ref, ow_ref, ob_ref,
             out_ref, idx_ref, part_ref):
    x = x_ref[...]
    loss_acc = jnp.float32(0.0)
    q_parts = []
    for g in range(_NUM_GROUPS):
        xg = x[:, g * _GROUP_DIM:(g + 1) * _GROUP_DIM]
        p = jnp.dot(xg, pw_ref[g], preferred_element_type=jnp.float32)
        p = p + pb_ref[g][None, :]
        c = cb_ref[g]
        cross2 = jax.lax.dot_general(
            p * jnp.float32(-2.0), c, (((1,), (1,)), ((), ())),
            preferred_element_type=jnp.float32)
        p2 = jnp.sum(p * p, axis=-1, keepdims=True)
        c2 = jnp.sum(c * c, axis=-1)
        dist = (p2 + cross2) + c2[None, :]
        minv = jnp.min(dist, axis=-1)
        loss_acc = loss_acc + jnp.sum(minv)
        eq = dist == minv[:, None]
        iota = jax.lax.broadcasted_iota(jnp.int32, dist.shape, 1)
        idx = jnp.min(jnp.where(eq, iota, _K), axis=-1)
        onehot = jnp.where(iota == idx[:, None], jnp.float32(1.0),
                           jnp.float32(0.0))
        qg = jnp.dot(onehot, c, preferred_element_type=jnp.float32)
        q_parts.append(qg)
        idx_ref[g, :] = idx
    q = jnp.concatenate(q_parts, axis=-1)
    out = jnp.dot(q, ow_ref[...], preferred_element_type=jnp.float32)
    out_ref[...] = out + ob_ref[...]
    part_ref[0, 0, 0] = loss_acc


@jax.jit
def _vq_call(x, codebooks, proj_w, proj_b, out_w, out_b2d):
    n = x.shape[0]
    grid = n // _TILE
    out, idx_gm, partials = pl.pallas_call(
        _vq_body,
        grid=(grid,),
        in_specs=[
            pl.BlockSpec((_TILE, _EMBED), lambda i: (i, 0)),
            pl.BlockSpec((_NUM_GROUPS, _K, _GROUP_DIM), lambda i: (0, 0, 0)),
            pl.BlockSpec((_NUM_GROUPS, _GROUP_DIM, _GROUP_DIM),
                         lambda i: (0, 0, 0)),
            pl.BlockSpec((_NUM_GROUPS, _GROUP_DIM), lambda i: (0, 0)),
            pl.BlockSpec((_EMBED, _EMBED), lambda i: (0, 0)),
            pl.BlockSpec((1, _EMBED), lambda i: (0, 0)),
        ],
        out_specs=[
            pl.BlockSpec((_TILE, _EMBED), lambda i: (i, 0)),
            pl.BlockSpec((_NUM_GROUPS, _TILE), lambda i: (0, i)),
            pl.BlockSpec((1, 1, 1), lambda i: (i, 0, 0),
                         memory_space=pltpu.SMEM),
        ],
        out_shape=[
            jax.ShapeDtypeStruct((n, _EMBED), jnp.float32),
            jax.ShapeDtypeStruct((_NUM_GROUPS, n), jnp.int32),
            jax.ShapeDtypeStruct((grid, 1, 1), jnp.float32),
        ],
    )(x, codebooks, proj_w, proj_b, out_w, out_b2d)
    return out, idx_gm, partials


def kernel(features, codebooks, proj_w, proj_b, out_w, out_b):
    b, s, e = features.shape
    x = features.reshape(b * s, e)
    out, idx_gm, partials = _vq_call(
        x, codebooks, proj_w, proj_b, out_w, out_b.reshape(1, e))
    quantized_features = out.reshape(b, s, e)
    indices = idx_gm.T.reshape(b, s, _NUM_GROUPS)
    scale = _BETA / (_NUM_GROUPS * b * s * _GROUP_DIM)
    total_commitment_loss = jnp.sum(partials) * scale
    return (quantized_features, indices, total_commitment_loss)
